# Initial kernel scaffold; baseline (speedup 1.0000x reference)
import jax
import jax.numpy as jnp
from jax.experimental import pallas as pl


def kernel(x1, edge_index1, batch1, cell, x2, edge_index2, batch2, W1, b1, W2, b2, W3, b3, Wg1, bg1, Wg2, bg2, R1w, R1b, R2w, R2b, R3w, R3b, F1w, F1b, F2w, F2b, Ow, Ob):
    raise NotImplementedError("write your pallas kernel here")


# speedup vs baseline: 1.3267x; 1.3267x over previous
import functools
import jax
import jax.numpy as jnp
from jax.experimental import pallas as pl
from jax.experimental.pallas import tpu as pltpu


def _gcn_conv(x, edge_index, W, b, dis):
    n = x.shape[0]
    h = x @ W
    src = edge_index[0]
    dst = edge_index[1]
    norm = dis[src] * dis[dst]
    msg = h[src] * norm[:, None]
    out = jnp.zeros((n, h.shape[1]), h.dtype).at[dst].add(msg)
    out = out + h * (dis * dis)[:, None]
    return out + b


def _branch(x, ei, batch, B, W1, b1, W2, b2, W3, b3):
    n = x.shape[0]
    deg = jnp.ones((n,), jnp.float32).at[ei[1]].add(1.0)
    dis = jax.lax.rsqrt(deg)
    h = jax.nn.relu(_gcn_conv(x, ei, W1, b1, dis))
    h = jax.nn.relu(_gcn_conv(h, ei, W2, b2, dis))
    h = jax.nn.relu(_gcn_conv(h, ei, W3, b3, dis))
    g = jax.ops.segment_max(h, batch, num_segments=B)
    g = jnp.where(jnp.isfinite(g), g, 0.0)
    return g


def _head_body(g1_ref, g2_ref, cell_ref,
               Wg1_ref, bg1_ref, Wg2_ref, bg2_ref,
               R1w_ref, R1b_ref, R2w_ref, R2b_ref, R3w_ref, R3b_ref,
               F1w_ref, F1b_ref, F2w_ref, F2b_ref, Ow_ref, Ob_ref,
               out_ref):
    def mlp_tail(g):
        g = jax.nn.relu(jnp.dot(g, Wg1_ref[...],
                                preferred_element_type=jnp.float32) + bg1_ref[...])
        return jnp.dot(g, Wg2_ref[...],
                       preferred_element_type=jnp.float32) + bg2_ref[...]

    g1 = mlp_tail(g1_ref[...])
    g2 = mlp_tail(g2_ref[...])
    cell = cell_ref[...]
    nrm = jnp.sqrt(jnp.sum(cell * cell, axis=1, keepdims=True))
    cv = cell / jnp.maximum(nrm, 1e-12)
    cv = jax.nn.relu(jnp.dot(cv, R1w_ref[...],
                             preferred_element_type=jnp.float32) + R1b_ref[...])
    cv = jax.nn.relu(jnp.dot(cv, R2w_ref[...],
                             preferred_element_type=jnp.float32) + R2b_ref[...])
    cv = jnp.dot(cv, R3w_ref[...],
                 preferred_element_type=jnp.float32) + R3b_ref[...]
    xc = jnp.concatenate([g1, g2, cv], axis=1)
    xc = jax.nn.relu(jnp.dot(xc, F1w_ref[...],
                             preferred_element_type=jnp.float32) + F1b_ref[...])
    xc = jax.nn.relu(jnp.dot(xc, F2w_ref[...],
                             preferred_element_type=jnp.float32) + F2b_ref[...])
    out_ref[...] = jnp.dot(xc, Ow_ref[...],
                           preferred_element_type=jnp.float32) + Ob_ref[...]


def kernel(x1, edge_index1, batch1, cell, x2, edge_index2, batch2, W1, b1, W2, b2, W3, b3, Wg1, bg1, Wg2, bg2, R1w, R1b, R2w, R2b, R3w, R3b, F1w, F1b, F2w, F2b, Ow, Ob):
    B = cell.shape[0]
    g1 = _branch(x1, edge_index1, batch1, B, W1, b1, W2, b2, W3, b3)
    g2 = _branch(x2, edge_index2, batch2, B, W1, b1, W2, b2, W3, b3)

    Ow_p = jnp.pad(Ow, ((0, 0), (0, 126)))
    Ob_p = jnp.pad(Ob, ((0, 126),))

    head = pl.pallas_call(
        _head_body,
        out_shape=jax.ShapeDtypeStruct((B, 128), jnp.float32),
    )
    out = head(g1, g2, cell, Wg1, bg1, Wg2, bg2,
               R1w, R1b, R2w, R2b, R3w, R3b,
               F1w, F1b, F2w, F2b, Ow_p, Ob_p)
    return out[:, :2]


# speedup vs baseline: 2.0172x; 1.5205x over previous
import functools
import jax
import jax.numpy as jnp
from jax import lax
from jax.experimental import pallas as pl
from jax.experimental.pallas import tpu as pltpu
from jax.experimental.pallas import tpu_sc as plsc

NC, NS = 2, 16
NPAD = 51200
EPAD = 819200
EROWS = EPAD // 128
NSEG = 144
SEGW = 320
POOL_FLAT = NSEG * SEGW

_MESH = plsc.VectorSubcoreMesh(
    core_axis_name="c", subcore_axis_name="s", num_cores=NC, num_subcores=NS)


@functools.partial(
    pl.kernel,
    out_type=jax.ShapeDtypeStruct((NPAD, 16), jnp.float32),
    mesh=_MESH,
    scratch_types=[
        pltpu.VMEM_SHARED((25616, 16), jnp.float32),
        pltpu.VMEM((4, 128), jnp.int32),
        pltpu.VMEM((128, 16), jnp.float32),
    ],
    compiler_params=pltpu.CompilerParams(use_tc_tiling_on_sc=False),
)
def _deg_kernel(dst2d, zeros16, ones16, out, acc, didx, ones_v):
    c = lax.axis_index("c")
    s = lax.axis_index("s")
    base = c * 25600
    pltpu.sync_copy(ones16, ones_v)
    pltpu.sync_copy(zeros16, acc.at[pl.ds(s * 1601, 1601)])
    plsc.subcore_barrier()

    def step(i, carry):
        row0 = s * (EROWS // NS) + i * 4
        pltpu.sync_copy(dst2d.at[pl.ds(row0, 4)], didx)
        for j in range(4):
            for k in range(8):
                d = didx[j, pl.ds(k * 16, 16)]
                l = d - base
                m = (l >= 0) & (l < 25600)
                didx[j, pl.ds(k * 16, 16)] = jnp.where(m, l, 25600)
        for j in range(4):
            pltpu.sync_copy(ones_v, acc.at[didx.at[j]], add=True)
        return carry

    lax.fori_loop(0, EROWS // NS // 4, step, None)
    plsc.subcore_barrier()
    pltpu.sync_copy(acc.at[pl.ds(s * 1600, 1600)],
                    out.at[pl.ds(base + s * 1600, 1600)])


def _make_agg(F, RANGE, NPASS):
    ACC = RANGE + 16
    ZR = ACC // NS
    DR = RANGE // NS
    G = 320 // F

    @functools.partial(
        pl.kernel,
        out_type=jax.ShapeDtypeStruct((NPAD, F), jnp.float32),
        mesh=_MESH,
        scratch_types=[
            pltpu.VMEM_SHARED((ACC, F), jnp.float32),
            pltpu.VMEM((G, 128), jnp.int32),
            pltpu.VMEM((G, 128), jnp.int32),
            pltpu.VMEM((G * 128, F), jnp.float32),
            pltpu.SemaphoreType.DMA,
        ],
        compiler_params=pltpu.CompilerParams(use_tc_tiling_on_sc=False),
    )
    def agg(src2d, dst2d, p_hbm, zeros, out, acc, sidx, didx, rows, sem):
        c = lax.axis_index("c")
        s = lax.axis_index("s")
        for r in range(NPASS):
            base = (c * NPASS + r) * RANGE
            pltpu.sync_copy(zeros, acc.at[pl.ds(s * ZR, ZR)])
            plsc.subcore_barrier()

            def step(i, carry):
                row0 = s * (EROWS // NS) + i * G
                pltpu.sync_copy(src2d.at[pl.ds(row0, G)], sidx)
                pltpu.sync_copy(dst2d.at[pl.ds(row0, G)], didx)
                for j in range(G):
                    for k in range(8):
                        d = didx[j, pl.ds(k * 16, 16)]
                        l = d - base
                        m = (l >= 0) & (l < RANGE)
                        didx[j, pl.ds(k * 16, 16)] = jnp.where(m, l, RANGE)
                descs = [
                    pltpu.async_copy(p_hbm.at[sidx.at[j]],
                                     rows.at[pl.ds(j * 128, 128)], sem)
                    for j in range(G)
                ]
                for d_ in descs:
                    d_.wait()
                for j in range(G):
                    pltpu.sync_copy(rows.at[pl.ds(j * 128, 128)],
                                    acc.at[didx.at[j]], add=True)
                return carry

            lax.fori_loop(0, EROWS // NS // G, step, None)
            plsc.subcore_barrier()
            pltpu.sync_copy(acc.at[pl.ds(s * DR, DR)],
                            out.at[pl.ds(base + s * DR, DR)])
            plsc.subcore_barrier()

    return agg


_agg80 = _make_agg(80, 12800, 2)
_agg160 = _make_agg(160, 6400, 4)


@functools.partial(
    pl.kernel,
    out_type=jax.ShapeDtypeStruct((NC, POOL_FLAT), jnp.float32),
    mesh=_MESH,
    scratch_types=[
        pltpu.VMEM_SHARED((NS, POOL_FLAT), jnp.float32),
        pltpu.VMEM((POOL_FLAT,), jnp.float32),
        pltpu.VMEM((1616,), jnp.int32),
        pltpu.VMEM((64, SEGW), jnp.float32),
        pltpu.VMEM((2880,), jnp.float32),
        pltpu.VMEM((2880,), jnp.float32),
    ],
    compiler_params=pltpu.CompilerParams(use_tc_tiling_on_sc=False),
)
def _pool_kernel(h3, batch32, out, shared, acc1, bids, hbuf, mbuf, tbuf):
    c = lax.axis_index("c")
    s = lax.axis_index("s")
    wid = c * NS + s

    def zi(i, carry):
        acc1[pl.ds(i * 16, 16)] = jnp.full((16,), -3.4e38, jnp.float32)
        return carry

    lax.fori_loop(0, POOL_FLAT // 16, zi, None)
    pltpu.sync_copy(batch32.at[wid], bids.at[pl.ds(0, 1600)])

    def chunk(k, carry):
        pltpu.sync_copy(h3.at[pl.ds(wid * 1600 + k * 64, 64)], hbuf)

        def row(r, carry2):
            seg = bids[pl.ds(k * 64 + r, 16)][0]
            off = seg * SEGW
            for cc in range(SEGW // 16):
                a = acc1[pl.ds(off + cc * 16, 16)]
                h = hbuf[r, pl.ds(cc * 16, 16)]
                acc1[pl.ds(off + cc * 16, 16)] = jnp.maximum(a, h)
            return carry2

        lax.fori_loop(0, 64, row, None)
        return carry

    lax.fori_loop(0, 25, chunk, None)
    pltpu.sync_copy(acc1, shared.at[s])
    plsc.subcore_barrier()

    off = s * 2880
    pltpu.sync_copy(shared.at[0, pl.ds(off, 2880)], mbuf)
    for q in range(1, NS):
        pltpu.sync_copy(shared.at[q, pl.ds(off, 2880)], tbuf)

        def mx(i, carry):
            mbuf[pl.ds(i * 16, 16)] = jnp.maximum(mbuf[pl.ds(i * 16, 16)],
                                                  tbuf[pl.ds(i * 16, 16)])
            return carry

        lax.fori_loop(0, 180, mx, None)
    pltpu.sync_copy(mbuf, out.at[c, pl.ds(off, 2880)])


def _prep_body(x_ref, deg_ref, o_ref):
    dis = lax.rsqrt(deg_ref[...][:, :1] + 1.0)
    o_ref[...] = dis * x_ref[...]


def _prep(x, deg16):
    F = x.shape[1]
    return pl.pallas_call(
        _prep_body,
        grid=(NPAD // 512,),
        in_specs=[pl.BlockSpec((512, F), lambda i: (i, 0)),
                  pl.BlockSpec((512, 16), lambda i: (i, 0))],
        out_specs=pl.BlockSpec((512, F), lambda i: (i, 0)),
        out_shape=jax.ShapeDtypeStruct((NPAD, F), jnp.float32),
    )(x, deg16)


def _layer_body(last, q_ref, p_ref, deg_ref, w_ref, b_ref, o_ref):
    dis = lax.rsqrt(deg_ref[...][:, :1] + 1.0)
    u = dis * (q_ref[...] + p_ref[...])
    h = jnp.maximum(
        jnp.dot(u, w_ref[...], preferred_element_type=jnp.float32)
        + b_ref[...], 0.0)
    o_ref[...] = h if last else dis * h


def _layer(q, p, deg16, W, b, last):
    Fi, Fo = W.shape
    return pl.pallas_call(
        functools.partial(_layer_body, last),
        grid=(NPAD // 512,),
        in_specs=[pl.BlockSpec((512, Fi), lambda i: (i, 0)),
                  pl.BlockSpec((512, Fi), lambda i: (i, 0)),
                  pl.BlockSpec((512, 16), lambda i: (i, 0)),
                  pl.BlockSpec((Fi, Fo), lambda i: (0, 0)),
                  pl.BlockSpec((1, Fo), lambda i: (0, 0))],
        out_specs=pl.BlockSpec((512, Fo), lambda i: (i, 0)),
        out_shape=jax.ShapeDtypeStruct((NPAD, Fo), jnp.float32),
    )(q, p, deg16, W, b.reshape(1, -1))


def _head_body(gp1_ref, gp2_ref, cell_ref,
               Wg1_ref, bg1_ref, Wg2_ref, bg2_ref,
               R1w_ref, R1b_ref, R2w_ref, R2b_ref, R3w_ref, R3b_ref,
               F1w_ref, F1b_ref, F2w_ref, F2b_ref, Ow_ref, Ob_ref,
               out_ref):
    def branch_tail(gp_ref):
        gp = gp_ref[...]
        g = jnp.maximum(gp[0], gp[1])
        g = jnp.where(g > -1e37, g, 0.0)
        g = jax.nn.relu(jnp.dot(g, Wg1_ref[...],
                                preferred_element_type=jnp.float32)
                        + bg1_ref[...])
        return jnp.dot(g, Wg2_ref[...],
                       preferred_element_type=jnp.float32) + bg2_ref[...]

    g1 = branch_tail(gp1_ref)
    g2 = branch_tail(gp2_ref)
    cell = cell_ref[...]
    nrm = jnp.sqrt(jnp.sum(cell * cell, axis=1, keepdims=True))
    cv = cell / jnp.maximum(nrm, 1e-12)
    cv = jax.nn.relu(jnp.dot(cv, R1w_ref[...],
                             preferred_element_type=jnp.float32) + R1b_ref[...])
    cv = jax.nn.relu(jnp.dot(cv, R2w_ref[...],
                             preferred_element_type=jnp.float32) + R2b_ref[...])
    cv = jnp.dot(cv, R3w_ref[...],
                 preferred_element_type=jnp.float32) + R3b_ref[...]
    xc = jnp.concatenate([g1, g2, cv], axis=1)
    xc = jax.nn.relu(jnp.dot(xc, F1w_ref[...],
                             preferred_element_type=jnp.float32) + F1b_ref[...])
    xc = jax.nn.relu(jnp.dot(xc, F2w_ref[...],
                             preferred_element_type=jnp.float32) + F2b_ref[...])
    out_ref[...] = jnp.dot(xc, Ow_ref[...],
                           preferred_element_type=jnp.float32) + Ob_ref[...]


def kernel(x1, edge_index1, batch1, cell, x2, edge_index2, batch2,
           W1, b1, W2, b2, W3, b3, Wg1, bg1, Wg2, bg2,
           R1w, R1b, R2w, R2b, R3w, R3b, F1w, F1b, F2w, F2b, Ow, Ob):
    N = x1.shape[0]
    E = edge_index1.shape[1]
    B = cell.shape[0]

    def padw(W, ri, ci):
        return jnp.pad(W, ((0, ri - W.shape[0]), (0, ci - W.shape[1])))

    W1p, b1p = padw(W1, 80, 80), jnp.pad(b1, (0, 2))
    W2p, b2p = padw(W2, 80, 160), jnp.pad(b2, (0, 4))
    W3p, b3p = padw(W3, 160, 320), jnp.pad(b3, (0, 8))
    Wg1p, bg1p = padw(Wg1, 320, 160), jnp.pad(bg1, (0, 4))
    Wg2p = jnp.pad(Wg2, ((0, 4), (0, 0)))

    zeros16 = jnp.zeros((1601, 16), jnp.float32)
    ones16 = jnp.ones((128, 16), jnp.float32)
    zeros80 = jnp.zeros((12816 // NS, 80), jnp.float32)
    zeros160 = jnp.zeros((6416 // NS, 160), jnp.float32)

    def branch(x, ei, batch):
        src = jnp.concatenate(
            [ei[0], jnp.zeros((EPAD - E,), jnp.int32)]).reshape(EROWS, 128)
        dst = jnp.concatenate(
            [ei[1], jnp.full((EPAD - E,), NPAD, jnp.int32)]).reshape(EROWS, 128)
        xp = jnp.pad(x, ((0, NPAD - N), (0, 80 - x.shape[1])))
        b32 = jnp.concatenate(
            [batch, jnp.full((NPAD - N,), B, jnp.int32)]).reshape(32, 1600)

        deg16 = _deg_kernel(dst, zeros16, ones16)
        p0 = _prep(xp, deg16)
        q1 = _agg80(src, dst, p0, zeros80)
        p1 = _layer(q1, p0, deg16, W1p, b1p, last=False)
        q2 = _agg80(src, dst, p1, zeros80)
        p2 = _layer(q2, p1, deg16, W2p, b2p, last=False)
        q3 = _agg160(src, dst, p2, zeros160)
        h3 = _layer(q3, p2, deg16, W3p, b3p, last=True)
        gp = _pool_kernel(h3, b32)
        return gp.reshape(NC, NSEG, SEGW)[:, :B, :]

    gp1 = branch(x1, edge_index1, batch1)
    gp2 = branch(x2, edge_index2, batch2)

    Ow_p = jnp.pad(Ow, ((0, 0), (0, 126)))
    Ob_p = jnp.pad(Ob, ((0, 126),))

    head = pl.pallas_call(
        _head_body,
        out_shape=jax.ShapeDtypeStruct((B, 128), jnp.float32),
    )
    out = head(gp1, gp2, cell, Wg1p, bg1p, Wg2p, bg2,
               R1w, R1b, R2w, R2b, R3w, R3b,
               F1w, F1b, F2w, F2b, Ow_p, Ob_p)
    return out[:, :2]


# speedup vs baseline: 3.8653x; 1.9162x over previous
import functools
import jax
import jax.numpy as jnp
from jax import lax
from jax.experimental import pallas as pl
from jax.experimental.pallas import tpu as pltpu
from jax.experimental.pallas import tpu_sc as plsc

NC, NS = 2, 16
NPAD = 51200
EPAD = 819200
EROWS = EPAD // 128
NSEG = 144
SEGW = 320
POOL_FLAT = NSEG * SEGW

_MESH = plsc.VectorSubcoreMesh(
    core_axis_name="c", subcore_axis_name="s", num_cores=NC, num_subcores=NS)


@functools.partial(
    pl.kernel,
    out_type=jax.ShapeDtypeStruct((NPAD, 16), jnp.float32),
    mesh=_MESH,
    scratch_types=[
        pltpu.VMEM_SHARED((25616, 16), jnp.float32),
        pltpu.VMEM((4, 128), jnp.int32),
        pltpu.VMEM((128, 16), jnp.float32),
    ],
    compiler_params=pltpu.CompilerParams(use_tc_tiling_on_sc=False),
)
def _deg_kernel(dst2d, zeros16, ones16, out, acc, didx, ones_v):
    c = lax.axis_index("c")
    s = lax.axis_index("s")
    base = c * 25600
    pltpu.sync_copy(ones16, ones_v)
    pltpu.sync_copy(zeros16, acc.at[pl.ds(s * 1601, 1601)])
    plsc.subcore_barrier()

    def step(i, carry):
        row0 = s * (EROWS // NS) + i * 4
        pltpu.sync_copy(dst2d.at[pl.ds(row0, 4)], didx)
        for j in range(4):
            for k in range(8):
                d = didx[j, pl.ds(k * 16, 16)]
                l = d - base
                m = (l >= 0) & (l < 25600)
                didx[j, pl.ds(k * 16, 16)] = jnp.where(m, l, 25600)
        for j in range(4):
            pltpu.sync_copy(ones_v, acc.at[didx.at[j]], add=True)
        return carry

    lax.fori_loop(0, EROWS // NS // 4, step, None)
    plsc.subcore_barrier()
    pltpu.sync_copy(acc.at[pl.ds(s * 1600, 1600)],
                    out.at[pl.ds(base + s * 1600, 1600)])


RANGE = 25600
ACC = RANGE + 16
ZR = ACC // NS
DR = RANGE // NS
FC = 40
AG = 10


def _make_agg(K):
    @functools.partial(
        pl.kernel,
        out_type=jax.ShapeDtypeStruct((K, NPAD, FC), jnp.float32),
        mesh=_MESH,
        scratch_types=[
            pltpu.VMEM_SHARED((ACC, FC), jnp.float32),
            pltpu.VMEM((AG, 128), jnp.int32),
            pltpu.VMEM((AG, 128), jnp.int32),
            pltpu.VMEM((AG * 128, FC), jnp.float32),
            pltpu.SemaphoreType.DMA,
        ],
        compiler_params=pltpu.CompilerParams(use_tc_tiling_on_sc=False),
    )
    def agg(src2d, dst2d, *rest):
        pks = rest[:K]
        zeros, out, acc, sidx, didx, rows, sem = rest[K:]
        c = lax.axis_index("c")
        s = lax.axis_index("s")
        base = c * RANGE
        for k in range(K):
            pltpu.sync_copy(zeros, acc.at[pl.ds(s * ZR, ZR)])
            plsc.subcore_barrier()

            def step(i, carry):
                row0 = s * (EROWS // NS) + i * AG
                pltpu.sync_copy(src2d.at[pl.ds(row0, AG)], sidx)
                pltpu.sync_copy(dst2d.at[pl.ds(row0, AG)], didx)
                for j in range(AG):
                    for q in range(8):
                        d = didx[j, pl.ds(q * 16, 16)]
                        l = d - base
                        m = (l >= 0) & (l < RANGE)
                        didx[j, pl.ds(q * 16, 16)] = jnp.where(m, l, RANGE)
                descs = [
                    pltpu.async_copy(pks[k].at[sidx.at[j]],
                                     rows.at[pl.ds(j * 128, 128)], sem)
                    for j in range(AG)
                ]
                for d_ in descs:
                    d_.wait()
                for j in range(AG):
                    pltpu.sync_copy(rows.at[pl.ds(j * 128, 128)],
                                    acc.at[didx.at[j]], add=True)
                return carry

            lax.fori_loop(0, EROWS // NS // AG, step, None)
            plsc.subcore_barrier()
            pltpu.sync_copy(acc.at[pl.ds(s * DR, DR)],
                            out.at[k, pl.ds(base + s * DR, DR)])
            plsc.subcore_barrier()

    return agg


_agg2 = _make_agg(2)
_agg4 = _make_agg(4)


@functools.partial(
    pl.kernel,
    out_type=jax.ShapeDtypeStruct((NC, POOL_FLAT), jnp.float32),
    mesh=_MESH,
    scratch_types=[
        pltpu.VMEM_SHARED((NS, POOL_FLAT), jnp.float32),
        pltpu.VMEM((POOL_FLAT,), jnp.float32),
        pltpu.VMEM((1616,), jnp.int32),
        pltpu.VMEM((64, SEGW), jnp.float32),
        pltpu.VMEM((2880,), jnp.float32),
        pltpu.VMEM((2880,), jnp.float32),
    ],
    compiler_params=pltpu.CompilerParams(use_tc_tiling_on_sc=False),
)
def _pool_kernel(h3, batch32, out, shared, acc1, bids, hbuf, mbuf, tbuf):
    c = lax.axis_index("c")
    s = lax.axis_index("s")
    wid = c * NS + s

    def zi(i, carry):
        acc1[pl.ds(i * 16, 16)] = jnp.full((16,), -3.4e38, jnp.float32)
        return carry

    lax.fori_loop(0, POOL_FLAT // 16, zi, None)
    pltpu.sync_copy(batch32.at[wid], bids.at[pl.ds(0, 1600)])

    def chunk(k, carry):
        pltpu.sync_copy(h3.at[pl.ds(wid * 1600 + k * 64, 64)], hbuf)

        def row(r, carry2):
            seg = bids[pl.ds(k * 64 + r, 16)][0]
            off = seg * SEGW
            for cc in range(SEGW // 16):
                a = acc1[pl.ds(off + cc * 16, 16)]
                h = hbuf[r, pl.ds(cc * 16, 16)]
                acc1[pl.ds(off + cc * 16, 16)] = jnp.maximum(a, h)
            return carry2

        lax.fori_loop(0, 64, row, None)
        return carry

    lax.fori_loop(0, 25, chunk, None)
    pltpu.sync_copy(acc1, shared.at[s])
    plsc.subcore_barrier()

    off = s * 2880
    pltpu.sync_copy(shared.at[0, pl.ds(off, 2880)], mbuf)
    for q in range(1, NS):
        pltpu.sync_copy(shared.at[q, pl.ds(off, 2880)], tbuf)

        def mx(i, carry):
            mbuf[pl.ds(i * 16, 16)] = jnp.maximum(mbuf[pl.ds(i * 16, 16)],
                                                  tbuf[pl.ds(i * 16, 16)])
            return carry

        lax.fori_loop(0, 180, mx, None)
    pltpu.sync_copy(mbuf, out.at[c, pl.ds(off, 2880)])


def _prep_body(x_ref, deg_ref, o_ref):
    dis = lax.rsqrt(deg_ref[...][:, :1] + 1.0)
    o_ref[...] = dis * x_ref[...]


def _prep(x, deg16):
    F = x.shape[1]
    return pl.pallas_call(
        _prep_body,
        grid=(NPAD // 512,),
        in_specs=[pl.BlockSpec((512, F), lambda i: (i, 0)),
                  pl.BlockSpec((512, 16), lambda i: (i, 0))],
        out_specs=pl.BlockSpec((512, F), lambda i: (i, 0)),
        out_shape=jax.ShapeDtypeStruct((NPAD, F), jnp.float32),
    )(x, deg16)


def _layer_body(last, q_ref, p_ref, deg_ref, w_ref, b_ref, o_ref):
    dis = lax.rsqrt(deg_ref[...][:, :1] + 1.0)
    u = dis * (q_ref[...] + p_ref[...])
    h = jnp.maximum(
        jnp.dot(u, w_ref[...], preferred_element_type=jnp.float32)
        + b_ref[...], 0.0)
    o_ref[...] = h if last else dis * h


def _layer(q, p, deg16, W, b, last):
    Fi, Fo = W.shape
    return pl.pallas_call(
        functools.partial(_layer_body, last),
        grid=(NPAD // 512,),
        in_specs=[pl.BlockSpec((512, Fi), lambda i: (i, 0)),
                  pl.BlockSpec((512, Fi), lambda i: (i, 0)),
                  pl.BlockSpec((512, 16), lambda i: (i, 0)),
                  pl.BlockSpec((Fi, Fo), lambda i: (0, 0)),
                  pl.BlockSpec((1, Fo), lambda i: (0, 0))],
        out_specs=pl.BlockSpec((512, Fo), lambda i: (i, 0)),
        out_shape=jax.ShapeDtypeStruct((NPAD, Fo), jnp.float32),
    )(q, p, deg16, W, b.reshape(1, -1))


def _head_body(gp1_ref, gp2_ref, cell_ref,
               Wg1_ref, bg1_ref, Wg2_ref, bg2_ref,
               R1w_ref, R1b_ref, R2w_ref, R2b_ref, R3w_ref, R3b_ref,
               F1w_ref, F1b_ref, F2w_ref, F2b_ref, Ow_ref, Ob_ref,
               out_ref):
    def branch_tail(gp_ref):
        gp = gp_ref[...]
        g = jnp.maximum(gp[0], gp[1])
        g = jnp.where(g > -1e37, g, 0.0)
        g = jax.nn.relu(jnp.dot(g, Wg1_ref[...],
                                preferred_element_type=jnp.float32)
                        + bg1_ref[...])
        return jnp.dot(g, Wg2_ref[...],
                       preferred_element_type=jnp.float32) + bg2_ref[...]

    g1 = branch_tail(gp1_ref)
    g2 = branch_tail(gp2_ref)
    cell = cell_ref[...]
    nrm = jnp.sqrt(jnp.sum(cell * cell, axis=1, keepdims=True))
    cv = cell / jnp.maximum(nrm, 1e-12)
    cv = jax.nn.relu(jnp.dot(cv, R1w_ref[...],
                             preferred_element_type=jnp.float32) + R1b_ref[...])
    cv = jax.nn.relu(jnp.dot(cv, R2w_ref[...],
                             preferred_element_type=jnp.float32) + R2b_ref[...])
    cv = jnp.dot(cv, R3w_ref[...],
                 preferred_element_type=jnp.float32) + R3b_ref[...]
    xc = jnp.concatenate([g1, g2, cv], axis=1)
    xc = jax.nn.relu(jnp.dot(xc, F1w_ref[...],
                             preferred_element_type=jnp.float32) + F1b_ref[...])
    xc = jax.nn.relu(jnp.dot(xc, F2w_ref[...],
                             preferred_element_type=jnp.float32) + F2b_ref[...])
    out_ref[...] = jnp.dot(xc, Ow_ref[...],
                           preferred_element_type=jnp.float32) + Ob_ref[...]


def kernel(x1, edge_index1, batch1, cell, x2, edge_index2, batch2,
           W1, b1, W2, b2, W3, b3, Wg1, bg1, Wg2, bg2,
           R1w, R1b, R2w, R2b, R3w, R3b, F1w, F1b, F2w, F2b, Ow, Ob):
    N = x1.shape[0]
    E = edge_index1.shape[1]
    B = cell.shape[0]

    def padw(W, ri, ci):
        return jnp.pad(W, ((0, ri - W.shape[0]), (0, ci - W.shape[1])))

    W1p, b1p = padw(W1, 80, 80), jnp.pad(b1, (0, 2))
    W2p, b2p = padw(W2, 80, 160), jnp.pad(b2, (0, 4))
    W3p, b3p = padw(W3, 160, 320), jnp.pad(b3, (0, 8))
    Wg1p, bg1p = padw(Wg1, 320, 160), jnp.pad(bg1, (0, 4))
    Wg2p = jnp.pad(Wg2, ((0, 4), (0, 0)))

    zeros16 = jnp.zeros((1601, 16), jnp.float32)
    ones16 = jnp.ones((128, 16), jnp.float32)
    zeros40 = jnp.zeros((ZR, FC), jnp.float32)

    def branch(x, ei, batch):
        src = jnp.concatenate(
            [ei[0], jnp.zeros((EPAD - E,), jnp.int32)]).reshape(EROWS, 128)
        dst = jnp.concatenate(
            [ei[1], jnp.full((EPAD - E,), NPAD, jnp.int32)]).reshape(EROWS, 128)
        xp = jnp.pad(x, ((0, NPAD - N), (0, 80 - x.shape[1])))
        b32 = jnp.concatenate(
            [batch, jnp.full((NPAD - N,), B, jnp.int32)]).reshape(32, 1600)

        def aggregate(p):
            K = p.shape[1] // FC
            kfn = _agg2 if K == 2 else _agg4
            chunks = [p[:, i * FC:(i + 1) * FC] for i in range(K)]
            qc = kfn(src, dst, *chunks, zeros40)
            return jnp.concatenate(list(qc), axis=1)

        deg16 = _deg_kernel(dst, zeros16, ones16)
        p0 = _prep(xp, deg16)
        q1 = aggregate(p0)
        p1 = _layer(q1, p0, deg16, W1p, b1p, last=False)
        q2 = aggregate(p1)
        p2 = _layer(q2, p1, deg16, W2p, b2p, last=False)
        q3 = aggregate(p2)
        h3 = _layer(q3, p2, deg16, W3p, b3p, last=True)
        gp = _pool_kernel(h3, b32)
        return gp.reshape(NC, NSEG, SEGW)[:, :B, :]

    gp1 = branch(x1, edge_index1, batch1)
    gp2 = branch(x2, edge_index2, batch2)

    Ow_p = jnp.pad(Ow, ((0, 0), (0, 126)))
    Ob_p = jnp.pad(Ob, ((0, 126),))

    head = pl.pallas_call(
        _head_body,
        out_shape=jax.ShapeDtypeStruct((B, 128), jnp.float32),
    )
    out = head(gp1, gp2, cell, Wg1p, bg1p, Wg2p, bg2,
               R1w, R1b, R2w, R2b, R3w, R3b,
               F1w, F1b, F2w, F2b, Ow_p, Ob_p)
    return out[:, :2]
